# baseline (speedup 1.0000x reference)
import jax
import jax.numpy as jnp
from jax.experimental import pallas as pl
from jax.experimental.pallas import tpu as pltpu

QP = 64
C = 64
NPARAM = 11
_H_ROWS = (0, 3, 6, 9)
_A_ROWS = (2, 5, 8)


def _transform_body(t_ref, o_ref):
    sp = [jax.nn.softplus(t_ref[:, i, :]) for i in (0, 3, 6, 9)]
    ta = [jnp.tanh(t_ref[:, i, :]) for i in (2, 5, 8)]
    b = [t_ref[:, i, :] for i in (1, 4, 7, 10)]
    o_ref[:, 0, :] = sp[0]
    o_ref[:, 1, :] = b[0]
    for layer in range(3):
        scale = sp[layer + 1] if layer < 2 else 0.5 * sp[3]
        o_ref[:, 3 * layer + 2, :] = scale
        o_ref[:, 3 * layer + 3, :] = ta[layer] * scale
        o_ref[:, 3 * layer + 4, :] = b[layer + 1] * (1.0 if layer < 2 else 0.5)


def _main_body(idx_ref, p_ref, x_ref, o_ref):
    del idx_ref
    c0 = pl.program_id(1) * x_ref.shape[1]

    def chan(c, _):
        y = x_ref[0, c]

        def s(i):
            return p_ref[0, i, c0 + c]

        y = y * s(0) + s(1)
        for layer in range(3):
            y = y * s(3 * layer + 2) + jnp.tanh(y) * s(3 * layer + 3) + s(3 * layer + 4)
        o_ref[0, c] = 0.5 * jnp.tanh(y) + 0.5
        return 0

    jax.lax.fori_loop(0, x_ref.shape[1], chan, 0, unroll=8)


@jax.jit
def kernel(x, index, h1, b1, a1, h2, b2, a2, h3, b3, a3, h4, b4):
    B, Cx, H, W = x.shape
    table = jnp.stack(
        [t.reshape(QP, C) for t in (h1, b1, a1, h2, b2, a2, h3, b3, a3, h4, b4)],
        axis=1,
    )

    ttable = pl.pallas_call(
        _transform_body,
        out_shape=jax.ShapeDtypeStruct((QP, NPARAM, C), x.dtype),
    )(table)

    CS = 2
    CB = Cx // CS
    grid_spec = pltpu.PrefetchScalarGridSpec(
        num_scalar_prefetch=1,
        grid=(B, CS),
        in_specs=[
            pl.BlockSpec(
                (1, NPARAM, C),
                lambda b, cs, idx: (idx[b], 0, 0),
                memory_space=pltpu.SMEM,
            ),
            pl.BlockSpec((1, CB, H, W), lambda b, cs, idx: (b, cs, 0, 0)),
        ],
        out_specs=pl.BlockSpec((1, CB, H, W), lambda b, cs, idx: (b, cs, 0, 0)),
    )
    return pl.pallas_call(
        _main_body,
        grid_spec=grid_spec,
        out_shape=jax.ShapeDtypeStruct(x.shape, x.dtype),
    )(index, ttable, x)


# speedup vs baseline: 1.0869x; 1.0869x over previous
import jax
import jax.numpy as jnp
from jax.experimental import pallas as pl
from jax.experimental.pallas import tpu as pltpu

QP = 64
C = 64
NPARAM = 11
_H_ROWS = (0, 3, 6, 9)
_A_ROWS = (2, 5, 8)


def _transform_body(t_ref, o_ref):
    sp = [jax.nn.softplus(t_ref[:, i, :]) for i in (0, 3, 6, 9)]
    ta = [jnp.tanh(t_ref[:, i, :]) for i in (2, 5, 8)]
    b = [t_ref[:, i, :] for i in (1, 4, 7, 10)]
    o_ref[:, 0, :] = sp[0]
    o_ref[:, 1, :] = b[0]
    for layer in range(3):
        scale = sp[layer + 1] if layer < 2 else 0.5 * sp[3]
        o_ref[:, 3 * layer + 2, :] = scale
        o_ref[:, 3 * layer + 3, :] = ta[layer] * scale
        o_ref[:, 3 * layer + 4, :] = b[layer + 1] * (1.0 if layer < 2 else 0.5)


def _main_body(idx_ref, p_ref, x_ref, o_ref):
    del idx_ref
    c0 = pl.program_id(1) * x_ref.shape[1]

    def chan(c, _):
        y = x_ref[0, c]

        def s(i):
            return p_ref[0, i, c0 + c]

        y = y * s(0) + s(1)
        for layer in range(3):
            y = y * s(3 * layer + 2) + jnp.tanh(y) * s(3 * layer + 3) + s(3 * layer + 4)
        o_ref[0, c] = 0.5 * jnp.tanh(y) + 0.5
        return 0

    jax.lax.fori_loop(0, x_ref.shape[1], chan, 0, unroll=8)


@jax.jit
def kernel(x, index, h1, b1, a1, h2, b2, a2, h3, b3, a3, h4, b4):
    B, Cx, H, W = x.shape
    table = jnp.stack(
        [t.reshape(QP, C) for t in (h1, b1, a1, h2, b2, a2, h3, b3, a3, h4, b4)],
        axis=1,
    )

    ttable = pl.pallas_call(
        _transform_body,
        out_shape=jax.ShapeDtypeStruct((QP, NPARAM, C), x.dtype),
    )(table)

    CS = 1
    CB = Cx // CS
    grid_spec = pltpu.PrefetchScalarGridSpec(
        num_scalar_prefetch=1,
        grid=(B, CS),
        in_specs=[
            pl.BlockSpec(
                (1, NPARAM, C),
                lambda b, cs, idx: (idx[b], 0, 0),
                memory_space=pltpu.SMEM,
            ),
            pl.BlockSpec((1, CB, H, W), lambda b, cs, idx: (b, cs, 0, 0)),
        ],
        out_specs=pl.BlockSpec((1, CB, H, W), lambda b, cs, idx: (b, cs, 0, 0)),
    )
    return pl.pallas_call(
        _main_body,
        grid_spec=grid_spec,
        out_shape=jax.ShapeDtypeStruct(x.shape, x.dtype),
    )(index, ttable, x)


# speedup vs baseline: 1.1023x; 1.0142x over previous
import jax
import jax.numpy as jnp
from jax.experimental import pallas as pl
from jax.experimental.pallas import tpu as pltpu

QP = 64
C = 64
NPARAM = 11
_H_ROWS = (0, 3, 6, 9)
_A_ROWS = (2, 5, 8)


def _transform_body(t_ref, o_ref):
    sp = [jax.nn.softplus(t_ref[:, i, :]) for i in (0, 3, 6, 9)]
    ta = [jnp.tanh(t_ref[:, i, :]) for i in (2, 5, 8)]
    b = [t_ref[:, i, :] for i in (1, 4, 7, 10)]
    o_ref[:, 0, :] = sp[0]
    o_ref[:, 1, :] = b[0]
    for layer in range(3):
        scale = sp[layer + 1] if layer < 2 else 0.5 * sp[3]
        o_ref[:, 3 * layer + 2, :] = scale
        o_ref[:, 3 * layer + 3, :] = ta[layer] * scale
        o_ref[:, 3 * layer + 4, :] = b[layer + 1] * (1.0 if layer < 2 else 0.5)


def _main_body(idx_ref, p_ref, x_ref, o_ref):
    del idx_ref
    c0 = pl.program_id(1) * x_ref.shape[1]

    def chan(c, _):
        y = x_ref[0, c]

        def s(i):
            return p_ref[0, i, c0 + c]

        y = y * s(0) + s(1)
        for layer in range(3):
            y = y * s(3 * layer + 2) + jnp.tanh(y) * s(3 * layer + 3) + s(3 * layer + 4)
        o_ref[0, c] = 0.5 * jnp.tanh(y) + 0.5
        return 0

    jax.lax.fori_loop(0, x_ref.shape[1], chan, 0, unroll=16)


@jax.jit
def kernel(x, index, h1, b1, a1, h2, b2, a2, h3, b3, a3, h4, b4):
    B, Cx, H, W = x.shape
    table = jnp.stack(
        [t.reshape(QP, C) for t in (h1, b1, a1, h2, b2, a2, h3, b3, a3, h4, b4)],
        axis=1,
    )

    ttable = pl.pallas_call(
        _transform_body,
        out_shape=jax.ShapeDtypeStruct((QP, NPARAM, C), x.dtype),
    )(table)

    CS = 1
    CB = Cx // CS
    grid_spec = pltpu.PrefetchScalarGridSpec(
        num_scalar_prefetch=1,
        grid=(B, CS),
        in_specs=[
            pl.BlockSpec(
                (1, NPARAM, C),
                lambda b, cs, idx: (idx[b], 0, 0),
                memory_space=pltpu.SMEM,
            ),
            pl.BlockSpec((1, CB, H, W), lambda b, cs, idx: (b, cs, 0, 0)),
        ],
        out_specs=pl.BlockSpec((1, CB, H, W), lambda b, cs, idx: (b, cs, 0, 0)),
    )
    return pl.pallas_call(
        _main_body,
        grid_spec=grid_spec,
        out_shape=jax.ShapeDtypeStruct(x.shape, x.dtype),
    )(index, ttable, x)


# speedup vs baseline: 1.1062x; 1.0035x over previous
import jax
import jax.numpy as jnp
from jax.experimental import pallas as pl
from jax.experimental.pallas import tpu as pltpu

QP = 64
C = 64
NPARAM = 11
_H_ROWS = (0, 3, 6, 9)
_A_ROWS = (2, 5, 8)


def _transform_body(t_ref, o_ref):
    sp = [jax.nn.softplus(t_ref[:, i, :]) for i in (0, 3, 6, 9)]
    ta = [jnp.tanh(t_ref[:, i, :]) for i in (2, 5, 8)]
    b = [t_ref[:, i, :] for i in (1, 4, 7, 10)]
    o_ref[:, 0, :] = sp[0]
    o_ref[:, 1, :] = b[0]
    for layer in range(3):
        scale = sp[layer + 1] if layer < 2 else 0.5 * sp[3]
        o_ref[:, 3 * layer + 2, :] = scale
        o_ref[:, 3 * layer + 3, :] = ta[layer] * scale
        o_ref[:, 3 * layer + 4, :] = b[layer + 1] * (1.0 if layer < 2 else 0.5)


def _main_body(idx_ref, p_ref, x_ref, o_ref):
    del idx_ref
    c0 = pl.program_id(1) * x_ref.shape[1]

    def chan(c, _):
        y = x_ref[0, c]

        def s(i):
            return p_ref[0, i, c0 + c]

        y = y * s(0) + s(1)
        for layer in range(3):
            y = y * s(3 * layer + 2) + jnp.tanh(y) * s(3 * layer + 3) + s(3 * layer + 4)
        o_ref[0, c] = 0.5 * jnp.tanh(y) + 0.5
        return 0

    jax.lax.fori_loop(0, x_ref.shape[1], chan, 0, unroll=32)


@jax.jit
def kernel(x, index, h1, b1, a1, h2, b2, a2, h3, b3, a3, h4, b4):
    B, Cx, H, W = x.shape
    table = jnp.stack(
        [t.reshape(QP, C) for t in (h1, b1, a1, h2, b2, a2, h3, b3, a3, h4, b4)],
        axis=1,
    )

    ttable = pl.pallas_call(
        _transform_body,
        out_shape=jax.ShapeDtypeStruct((QP, NPARAM, C), x.dtype),
    )(table)

    CS = 1
    CB = Cx // CS
    grid_spec = pltpu.PrefetchScalarGridSpec(
        num_scalar_prefetch=1,
        grid=(B, CS),
        in_specs=[
            pl.BlockSpec(
                (1, NPARAM, C),
                lambda b, cs, idx: (idx[b], 0, 0),
                memory_space=pltpu.SMEM,
            ),
            pl.BlockSpec((1, CB, H, W), lambda b, cs, idx: (b, cs, 0, 0)),
        ],
        out_specs=pl.BlockSpec((1, CB, H, W), lambda b, cs, idx: (b, cs, 0, 0)),
    )
    return pl.pallas_call(
        _main_body,
        grid_spec=grid_spec,
        out_shape=jax.ShapeDtypeStruct(x.shape, x.dtype),
    )(index, ttable, x)
